# Initial kernel scaffold; baseline (speedup 1.0000x reference)
import jax
import jax.numpy as jnp
from jax.experimental import pallas as pl


def kernel(node_in, node_embed, edge_sh, edge_length_embedding, edge_src, edge_dst, batch, params):
    raise NotImplementedError("write your pallas kernel here")


# speedup vs baseline: 4.6669x; 4.6669x over previous
import functools

import jax
import jax.numpy as jnp
from jax import lax
from jax.experimental import pallas as pl
from jax.experimental.pallas import tpu as pltpu
from jax.experimental.pallas import tpu_sc as plsc

_N = 10000
_NP = 10240
_E = 160000
_D = 480
_DP = 512
_H = 4
_DLEN = 128

_NC = 2
_NS = 16
_GB = 80

_F32 = jnp.float32


def _ln(x, g, b):
    mu = jnp.mean(x, axis=-1, keepdims=True)
    var = jnp.mean((x - mu) ** 2, axis=-1, keepdims=True)
    return g * (x - mu) * lax.rsqrt(var + 1e-6) + b


def _silu(x):
    return x * jax.nn.sigmoid(x)


def _sc_gather(table, idx_all):
    mesh = plsc.VectorSubcoreMesh(core_axis_name="c", subcore_axis_name="s")
    rows_per_w = (2 * _E) // (_NC * _NS)
    iters = rows_per_w // _GB

    @functools.partial(
        pl.kernel,
        mesh=mesh,
        out_type=jax.ShapeDtypeStruct((2 * _E, _DP), _F32),
        scratch_types=[
            pltpu.VMEM((_GB,), jnp.int32),
            pltpu.VMEM((_GB, _DP), _F32),
            pltpu.SemaphoreType.DMA,
        ],
    )
    def gather_k(tbl_hbm, idx_hbm, out_hbm, idx_v, rows_v, sem):
        c = lax.axis_index("c")
        s = lax.axis_index("s")
        wid = s * _NC + c
        base = wid * rows_per_w

        def step(i, carry):
            r0 = base + i * _GB
            pltpu.sync_copy(idx_hbm.at[pl.ds(r0, _GB)], idx_v)
            pltpu.async_copy(tbl_hbm.at[idx_v], rows_v, sem).wait()
            pltpu.sync_copy(rows_v, out_hbm.at[pl.ds(r0, _GB)])
            return carry

        lax.fori_loop(0, iters, step, 0)

    return gather_k(table, idx_all)


def _tc_alpha(ele, p):
    be = 2000
    grid = _E // be

    def body(ele_ref, a1w, a1b, a1g, a1bb, a2w, a2b, a2g, a2bb, a3w, a3b,
             alpha_ref, g_ref):
        a = jnp.dot(ele_ref[...], a1w[...], preferred_element_type=_F32) + a1b[...]
        a = _silu(_ln(a, a1g[...], a1bb[...]))
        a = jnp.dot(a, a2w[...], preferred_element_type=_F32) + a2b[...]
        a = _silu(_ln(a, a2g[...], a2bb[...]))
        al8 = jnp.dot(a, a3w[...], preferred_element_type=_F32) + a3b[...]
        alpha_ref[...] = al8
        bm = jnp.max(al8[:, :_H])
        prev = jnp.where(pl.program_id(0) == 0, -3.0e38, g_ref[0, 0])
        g_ref[0, 0] = jnp.maximum(prev, bm)

    row = lambda v: v.reshape(1, -1)
    a3w8 = jnp.pad(p["a3"]["w"], ((0, 0), (0, 8 - _H)))
    a3b8 = row(jnp.pad(p["a3"]["b"], (0, 8 - _H)))
    weights = [
        p["a1"]["w"], row(p["a1"]["b"]), row(p["a1g"]), row(p["a1b"]),
        p["a2"]["w"], row(p["a2"]["b"]), row(p["a2g"]), row(p["a2b"]),
        a3w8, a3b8,
    ]
    in_specs = [pl.BlockSpec((be, _DLEN), lambda i: (i, 0))] + [
        pl.BlockSpec(w.shape, lambda i, nd=w.ndim: (0,) * nd) for w in weights
    ]
    alpha8, gmax = pl.pallas_call(
        body,
        grid=(grid,),
        in_specs=in_specs,
        out_specs=[
            pl.BlockSpec((be, 8), lambda i: (i, 0)),
            pl.BlockSpec((1, 1), lambda i: (0, 0), memory_space=pltpu.SMEM),
        ],
        out_shape=[
            jax.ShapeDtypeStruct((_E, 8), _F32),
            jax.ShapeDtypeStruct((1, 1), _F32),
        ],
    )(ele, *weights)
    return alpha8, gmax


def _tc_p(alpha8, gmax):
    be = 4000
    grid = _E // be

    def body(al_ref, g_ref, p_ref):
        p_ref[...] = jnp.exp(al_ref[...] - g_ref[0, 0])

    return pl.pallas_call(
        body,
        grid=(grid,),
        in_specs=[
            pl.BlockSpec((be, 8), lambda i: (i, 0)),
            pl.BlockSpec((1, 1), lambda i: (0, 0), memory_space=pltpu.SMEM),
        ],
        out_specs=pl.BlockSpec((be, 8), lambda i: (i, 0)),
        out_shape=jax.ShapeDtypeStruct((_E, 8), _F32),
    )(alpha8, gmax)


def _tc_edges(xsxd, shp, ele, alpha8, p8, weights):
    be = 640
    grid = _E // be

    def body(xs_ref, xd_ref, shp_ref, ele_ref, al_ref, p_ref,
             wp1_r, wp2_r, wp3_r, bp_r, wt1_r, wt2_r, wt3_r, bt_r,
             lew_r, leb_r, s1w_r, s1b_r, s1g_r, s1bb_r,
             s2w_r, s2b_r, s2g_r, s2bb_r, s3w_r, s3b_r,
             pv_ref, esc_ref):
        xs = xs_ref[...]
        xd = xd_ref[...]
        ele_b = ele_ref[...]
        msg = (jnp.dot(xs, wp1_r[...], preferred_element_type=_F32)
               + jnp.dot(xd, wp2_r[...], preferred_element_type=_F32)
               + jnp.dot(ele_b, wp3_r[...], preferred_element_type=_F32)
               + bp_r[...])
        shp_b = shp_ref[...]
        al = al_ref[...]
        pw = p_ref[...]
        es = jnp.zeros((be, 64), _F32) + leb_r[...]
        for h in range(_H):
            vh = (jnp.dot(msg, wt1_r[h], preferred_element_type=_F32)
                  + jnp.dot(shp_b, wt2_r[h], preferred_element_type=_F32)
                  + jnp.dot(ele_b, wt3_r[h], preferred_element_type=_F32)
                  + bt_r[h])
            eo = vh * al[:, h:h + 1]
            es = es + jnp.dot(eo, lew_r[h], preferred_element_type=_F32)
            pv_ref[:, h, :] = vh * pw[:, h:h + 1]
        es = _silu(_ln(jnp.dot(es, s1w_r[...], preferred_element_type=_F32)
                       + s1b_r[...], s1g_r[...], s1bb_r[...]))
        es = _silu(_ln(jnp.dot(es, s2w_r[...], preferred_element_type=_F32)
                       + s2b_r[...], s2g_r[...], s2bb_r[...]))
        esc_ref[...] = jnp.dot(es, s3w_r[...], preferred_element_type=_F32) + s3b_r[...]

    in_specs = [
        pl.BlockSpec((be, _DP), lambda i: (i, 0)),
        pl.BlockSpec((be, _DP), lambda i, g=grid: (i + g, 0)),
        pl.BlockSpec((be, 16), lambda i: (i, 0)),
        pl.BlockSpec((be, _DLEN), lambda i: (i, 0)),
        pl.BlockSpec((be, 8), lambda i: (i, 0)),
        pl.BlockSpec((be, 8), lambda i: (i, 0)),
    ] + [pl.BlockSpec(w.shape, lambda i, nd=w.ndim: (0,) * nd) for w in weights]

    pv, esc = pl.pallas_call(
        body,
        grid=(grid,),
        in_specs=in_specs,
        out_specs=[
            pl.BlockSpec((be, _H, _DP), lambda i: (i, 0, 0)),
            pl.BlockSpec((be, 32), lambda i: (i, 0)),
        ],
        out_shape=[
            jax.ShapeDtypeStruct((_E, _H, _DP), _F32),
            jax.ShapeDtypeStruct((_E, 32), _F32),
        ],
    )(xsxd, xsxd, shp, ele, alpha8, p8, *weights)
    return pv, esc


def _sc_node_scatter(pv2, dst, zeros128):
    mesh = plsc.VectorSubcoreMesh(core_axis_name="c", subcore_axis_name="s")
    e_per_t = _E // _NS
    iters = e_per_t // _GB
    cw = 128
    chunks_per_sc = (_H * _DP) // cw // _NC
    rows_per_t = _NP // _NS

    @functools.partial(
        pl.kernel,
        mesh=mesh,
        out_type=jax.ShapeDtypeStruct((_NP, _H * _DP), _F32),
        scratch_types=[
            pltpu.VMEM_SHARED((_NP, cw), _F32),
            pltpu.VMEM((_GB,), jnp.int32),
            pltpu.VMEM((_GB, cw), _F32),
        ],
    )
    def nf_k(pv_hbm, dst_hbm, z_hbm, out_hbm, acc, idx_v, buf):
        c = lax.axis_index("c")
        s = lax.axis_index("s")
        r0 = s * rows_per_t
        for j in range(chunks_per_sc):
            co = (c * chunks_per_sc + j) * cw
            pltpu.sync_copy(z_hbm, acc.at[pl.ds(r0, rows_per_t)])
            plsc.subcore_barrier()

            def step(i, carry):
                e0 = s * e_per_t + i * _GB
                pltpu.sync_copy(dst_hbm.at[pl.ds(e0, _GB)], idx_v)
                pltpu.sync_copy(pv_hbm.at[pl.ds(e0, _GB), pl.ds(co, cw)], buf)
                pltpu.sync_copy(buf, acc.at[idx_v], add=True)
                return carry

            lax.fori_loop(0, iters, step, 0)
            plsc.subcore_barrier()
            pltpu.sync_copy(acc.at[pl.ds(r0, rows_per_t)],
                            out_hbm.at[pl.ds(r0, rows_per_t), pl.ds(co, cw)])
            plsc.subcore_barrier()

    return nf_k(pv2, dst, zeros128)


def _tc_nodes(nf3, node_in, wl1h, wl2, bl):
    bn = 400
    grid = _N // bn

    def body(nf_ref, x_ref, wl1_r, wl2_r, bl_r, out_ref):
        acc = jnp.dot(x_ref[...], wl2_r[...], preferred_element_type=_F32) + bl_r[...]
        for h in range(_H):
            sden = nf_ref[:, h, _D:_D + 1]
            inv = jnp.where(sden > 0, 1.0 / sden, 0.0)
            acc = acc + jnp.dot(nf_ref[:, h, :] * inv,
                                wl1_r[h], preferred_element_type=_F32)
        out_ref[...] = acc

    weights = [wl1h, wl2, bl]
    in_specs = [
        pl.BlockSpec((bn, _H, _DP), lambda i: (i, 0, 0)),
        pl.BlockSpec((bn, _D), lambda i: (i, 0)),
    ] + [pl.BlockSpec(w.shape, lambda i, nd=w.ndim: (0,) * nd) for w in weights]

    return pl.pallas_call(
        body,
        grid=(grid,),
        in_specs=in_specs,
        out_specs=pl.BlockSpec((bn, _D), lambda i: (i, 0)),
        out_shape=jax.ShapeDtypeStruct((_N, _D), _F32),
    )(nf3, node_in, *weights)


def kernel(node_in, node_embed, edge_sh, edge_length_embedding, edge_src,
           edge_dst, batch, params):
    p = params
    ele = edge_length_embedding
    src = edge_src.astype(jnp.int32)
    dst = edge_dst.astype(jnp.int32)
    idx_all = jnp.concatenate([src, dst])
    row = lambda v: v.reshape(1, -1)

    node_pad = jnp.pad(node_in, ((0, 0), (0, _DP - _D)))
    wpre = p["pre"]["w"]
    padk = lambda w: jnp.pad(w, ((0, _DP - _D), (0, 0)))
    wp1, wp2, wp3 = padk(wpre[:_D]), padk(wpre[_D:2 * _D]), wpre[2 * _D:]
    bp = row(p["pre"]["b"])

    def padh(w3):
        return jnp.pad(w3.transpose(1, 0, 2), ((0, 0), (0, 0), (0, _DP - _D)))

    wtp2 = p["tp2"]["w"]
    wt1 = padh(wtp2[:_D].reshape(_D, _H, _D))
    wt2 = padh(jnp.pad(wtp2[_D:_D + 9], ((0, 7), (0, 0))).reshape(16, _H, _D))
    wt3 = padh(wtp2[_D + 9:].reshape(_DLEN, _H, _D))
    bt = jnp.pad(p["tp2"]["b"].reshape(_H, 1, _D), ((0, 0), (0, 0), (0, _DP - _D)))
    bt = bt.at[:, :, _D].set(1.0)
    lewh = jnp.pad(p["le"]["w"].reshape(_H, _D, 64), ((0, 0), (0, _DP - _D), (0, 0)))

    shp = jnp.pad(edge_sh, ((0, 0), (0, 7)))

    gath = _sc_gather(node_pad, idx_all)
    alpha8, gmax = _tc_alpha(ele, p)
    p8 = _tc_p(alpha8, gmax)
    heavy_weights = [
        wp1, wp2, wp3, bp, wt1, wt2, wt3, bt, lewh, row(p["le"]["b"]),
        p["s1"]["w"], row(p["s1"]["b"]), row(p["s1g"]), row(p["s1b"]),
        p["s2"]["w"], row(p["s2"]["b"]), row(p["s2g"]), row(p["s2b"]),
        p["s3"]["w"], row(p["s3"]["b"]),
    ]
    pv, esc = _tc_edges(gath, shp, ele, alpha8, p8, heavy_weights)
    pv2 = pv.reshape(_E, _H * _DP)
    zeros128 = jnp.zeros((_NP // _NS, 128), _F32)
    nf2 = _sc_node_scatter(pv2, dst, zeros128)
    nf3 = nf2.reshape(_NP, _H, _DP)
    wl = p["lin"]["w"]
    wl1h = jnp.pad(wl[:_H * _D].reshape(_H, _D, _D), ((0, 0), (0, _DP - _D), (0, 0)))
    node_out = _tc_nodes(nf3, node_in, wl1h, wl[_H * _D:], row(p["lin"]["b"]))
    return node_out, esc


# speedup vs baseline: 5.9002x; 1.2643x over previous
import functools

import jax
import jax.numpy as jnp
from jax import lax
from jax.experimental import pallas as pl
from jax.experimental.pallas import tpu as pltpu
from jax.experimental.pallas import tpu_sc as plsc

_N = 10000
_NP = 10240
_E = 160000
_D = 480
_DP = 512
_H = 4
_DLEN = 128

_NC = 2
_NS = 16
_GB = 80

_F32 = jnp.float32


def _ln(x, g, b):
    mu = jnp.mean(x, axis=-1, keepdims=True)
    var = jnp.mean((x - mu) ** 2, axis=-1, keepdims=True)
    return g * (x - mu) * lax.rsqrt(var + 1e-6) + b


def _silu(x):
    return x * jax.nn.sigmoid(x)


def _sc_gather(table, idx_all):
    mesh = plsc.VectorSubcoreMesh(core_axis_name="c", subcore_axis_name="s")
    rows_per_w = (2 * _E) // (_NC * _NS)
    iters = rows_per_w // _GB

    @functools.partial(
        pl.kernel,
        mesh=mesh,
        out_type=jax.ShapeDtypeStruct((2 * _E, _DP), _F32),
        scratch_types=[
            pltpu.VMEM((_GB,), jnp.int32),
            pltpu.VMEM((_GB, _DP), _F32),
            pltpu.SemaphoreType.DMA,
        ],
    )
    def gather_k(tbl_hbm, idx_hbm, out_hbm, idx_v, rows_v, sem):
        c = lax.axis_index("c")
        s = lax.axis_index("s")
        wid = s * _NC + c
        base = wid * rows_per_w

        def step(i, carry):
            r0 = base + i * _GB
            pltpu.sync_copy(idx_hbm.at[pl.ds(r0, _GB)], idx_v)
            pltpu.async_copy(tbl_hbm.at[idx_v], rows_v, sem).wait()
            pltpu.sync_copy(rows_v, out_hbm.at[pl.ds(r0, _GB)])
            return carry

        lax.fori_loop(0, iters, step, 0)

    return gather_k(table, idx_all)


def _tc_alpha(ele, p):
    be = 2000
    grid = _E // be

    def body(ele_ref, a1w, a1b, a1g, a1bb, a2w, a2b, a2g, a2bb, a3w, a3b,
             alpha_ref, g_ref):
        a = jnp.dot(ele_ref[...], a1w[...], preferred_element_type=_F32) + a1b[...]
        a = _silu(_ln(a, a1g[...], a1bb[...]))
        a = jnp.dot(a, a2w[...], preferred_element_type=_F32) + a2b[...]
        a = _silu(_ln(a, a2g[...], a2bb[...]))
        al8 = jnp.dot(a, a3w[...], preferred_element_type=_F32) + a3b[...]
        alpha_ref[...] = al8
        bm = jnp.max(al8[:, :_H])
        prev = jnp.where(pl.program_id(0) == 0, -3.0e38, g_ref[0, 0])
        g_ref[0, 0] = jnp.maximum(prev, bm)

    row = lambda v: v.reshape(1, -1)
    a3w8 = jnp.pad(p["a3"]["w"], ((0, 0), (0, 8 - _H)))
    a3b8 = row(jnp.pad(p["a3"]["b"], (0, 8 - _H)))
    weights = [
        p["a1"]["w"], row(p["a1"]["b"]), row(p["a1g"]), row(p["a1b"]),
        p["a2"]["w"], row(p["a2"]["b"]), row(p["a2g"]), row(p["a2b"]),
        a3w8, a3b8,
    ]
    in_specs = [pl.BlockSpec((be, _DLEN), lambda i: (i, 0))] + [
        pl.BlockSpec(w.shape, lambda i, nd=w.ndim: (0,) * nd) for w in weights
    ]
    alpha8, gmax = pl.pallas_call(
        body,
        grid=(grid,),
        in_specs=in_specs,
        out_specs=[
            pl.BlockSpec((be, 8), lambda i: (i, 0)),
            pl.BlockSpec((1, 1), lambda i: (0, 0), memory_space=pltpu.SMEM),
        ],
        out_shape=[
            jax.ShapeDtypeStruct((_E, 8), _F32),
            jax.ShapeDtypeStruct((1, 1), _F32),
        ],
    )(ele, *weights)
    return alpha8, gmax


def _tc_p(alpha8, gmax):
    be = 4000
    grid = _E // be

    def body(al_ref, g_ref, p_ref):
        p_ref[...] = jnp.exp(al_ref[...] - g_ref[0, 0])

    return pl.pallas_call(
        body,
        grid=(grid,),
        in_specs=[
            pl.BlockSpec((be, 8), lambda i: (i, 0)),
            pl.BlockSpec((1, 1), lambda i: (0, 0), memory_space=pltpu.SMEM),
        ],
        out_specs=pl.BlockSpec((be, 8), lambda i: (i, 0)),
        out_shape=jax.ShapeDtypeStruct((_E, 8), _F32),
    )(alpha8, gmax)


def _tc_edges(xsxd, shp, ele, alpha8, p8, weights):
    be = 640
    grid = _E // be

    def body(xs_ref, xd_ref, shp_ref, ele_ref, al_ref, p_ref,
             wp1_r, wp2_r, wp3_r, bp_r, wt1_r, wt2_r, wt3_r, bt_r,
             lew_r, leb_r, s1w_r, s1b_r, s1g_r, s1bb_r,
             s2w_r, s2b_r, s2g_r, s2bb_r, s3w_r, s3b_r,
             pv_ref, esc_ref):
        xs = xs_ref[...]
        xd = xd_ref[...]
        ele_b = ele_ref[...]
        msg = (jnp.dot(xs, wp1_r[...], preferred_element_type=_F32)
               + jnp.dot(xd, wp2_r[...], preferred_element_type=_F32)
               + jnp.dot(ele_b, wp3_r[...], preferred_element_type=_F32)
               + bp_r[...])
        shp_b = shp_ref[...]
        al = al_ref[...]
        pw = p_ref[...]
        es = jnp.zeros((be, 64), _F32) + leb_r[...]
        for h in range(_H):
            vh = (jnp.dot(msg, wt1_r[h], preferred_element_type=_F32)
                  + jnp.dot(shp_b, wt2_r[h], preferred_element_type=_F32)
                  + jnp.dot(ele_b, wt3_r[h], preferred_element_type=_F32)
                  + bt_r[h])
            eo = vh * al[:, h:h + 1]
            es = es + jnp.dot(eo, lew_r[h], preferred_element_type=_F32)
            pv_ref[:, h * _DP:(h + 1) * _DP] = vh * pw[:, h:h + 1]
        es = _silu(_ln(jnp.dot(es, s1w_r[...], preferred_element_type=_F32)
                       + s1b_r[...], s1g_r[...], s1bb_r[...]))
        es = _silu(_ln(jnp.dot(es, s2w_r[...], preferred_element_type=_F32)
                       + s2b_r[...], s2g_r[...], s2bb_r[...]))
        esc_ref[...] = jnp.dot(es, s3w_r[...], preferred_element_type=_F32) + s3b_r[...]

    in_specs = [
        pl.BlockSpec((be, _DP), lambda i: (i, 0)),
        pl.BlockSpec((be, _DP), lambda i, g=grid: (i + g, 0)),
        pl.BlockSpec((be, 16), lambda i: (i, 0)),
        pl.BlockSpec((be, _DLEN), lambda i: (i, 0)),
        pl.BlockSpec((be, 8), lambda i: (i, 0)),
        pl.BlockSpec((be, 8), lambda i: (i, 0)),
    ] + [pl.BlockSpec(w.shape, lambda i, nd=w.ndim: (0,) * nd) for w in weights]

    pv, esc = pl.pallas_call(
        body,
        grid=(grid,),
        in_specs=in_specs,
        out_specs=[
            pl.BlockSpec((be, _H * _DP), lambda i: (i, 0)),
            pl.BlockSpec((be, 32), lambda i: (i, 0)),
        ],
        out_shape=[
            jax.ShapeDtypeStruct((_E, _H * _DP), _F32),
            jax.ShapeDtypeStruct((_E, 32), _F32),
        ],
    )(xsxd, xsxd, shp, ele, alpha8, p8, *weights)
    return pv, esc


def _sc_node_scatter(pv2, dst, zeros128):
    mesh = plsc.VectorSubcoreMesh(core_axis_name="c", subcore_axis_name="s")
    e_per_t = _E // _NS
    iters = e_per_t // _GB
    cw = 128
    chunks_per_sc = (_H * _DP) // cw // _NC
    rows_per_t = _NP // _NS

    @functools.partial(
        pl.kernel,
        mesh=mesh,
        out_type=jax.ShapeDtypeStruct((_NP, _H * _DP), _F32),
        scratch_types=[
            pltpu.VMEM_SHARED((_NP, cw), _F32),
            pltpu.VMEM((_GB,), jnp.int32),
            pltpu.VMEM((_GB, cw), _F32),
        ],
    )
    def nf_k(pv_hbm, dst_hbm, z_hbm, out_hbm, acc, idx_v, buf):
        c = lax.axis_index("c")
        s = lax.axis_index("s")
        r0 = s * rows_per_t
        for j in range(chunks_per_sc):
            co = (c * chunks_per_sc + j) * cw
            pltpu.sync_copy(z_hbm, acc.at[pl.ds(r0, rows_per_t)])
            plsc.subcore_barrier()

            def step(i, carry):
                e0 = s * e_per_t + i * _GB
                pltpu.sync_copy(dst_hbm.at[pl.ds(e0, _GB)], idx_v)
                pltpu.sync_copy(pv_hbm.at[pl.ds(e0, _GB), pl.ds(co, cw)], buf)
                pltpu.sync_copy(buf, acc.at[idx_v], add=True)
                return carry

            lax.fori_loop(0, iters, step, 0)
            plsc.subcore_barrier()
            pltpu.sync_copy(acc.at[pl.ds(r0, rows_per_t)],
                            out_hbm.at[pl.ds(r0, rows_per_t), pl.ds(co, cw)])
            plsc.subcore_barrier()

    return nf_k(pv2, dst, zeros128)


def _tc_nodes(nf2, node_in, wl1h, wl2, bl):
    bn = 400
    grid = _N // bn

    def body(nf_ref, x_ref, wl1_r, wl2_r, bl_r, out_ref):
        acc = jnp.dot(x_ref[...], wl2_r[...], preferred_element_type=_F32) + bl_r[...]
        for h in range(_H):
            nh = nf_ref[:, h * _DP:(h + 1) * _DP]
            sden = nh[:, _D:_D + 1]
            inv = jnp.where(sden > 0, 1.0 / sden, 0.0)
            acc = acc + jnp.dot(nh * inv, wl1_r[h], preferred_element_type=_F32)
        out_ref[...] = acc

    weights = [wl1h, wl2, bl]
    in_specs = [
        pl.BlockSpec((bn, _H * _DP), lambda i: (i, 0)),
        pl.BlockSpec((bn, _D), lambda i: (i, 0)),
    ] + [pl.BlockSpec(w.shape, lambda i, nd=w.ndim: (0,) * nd) for w in weights]

    return pl.pallas_call(
        body,
        grid=(grid,),
        in_specs=in_specs,
        out_specs=pl.BlockSpec((bn, _D), lambda i: (i, 0)),
        out_shape=jax.ShapeDtypeStruct((_N, _D), _F32),
    )(nf2, node_in, *weights)


def kernel(node_in, node_embed, edge_sh, edge_length_embedding, edge_src,
           edge_dst, batch, params):
    p = params
    ele = edge_length_embedding
    src = edge_src.astype(jnp.int32)
    dst = edge_dst.astype(jnp.int32)
    idx_all = jnp.concatenate([src, dst])
    row = lambda v: v.reshape(1, -1)

    node_pad = jnp.pad(node_in, ((0, 0), (0, _DP - _D)))
    wpre = p["pre"]["w"]
    padk = lambda w: jnp.pad(w, ((0, _DP - _D), (0, 0)))
    wp1, wp2, wp3 = padk(wpre[:_D]), padk(wpre[_D:2 * _D]), wpre[2 * _D:]
    bp = row(p["pre"]["b"])

    def padh(w3):
        return jnp.pad(w3.transpose(1, 0, 2), ((0, 0), (0, 0), (0, _DP - _D)))

    wtp2 = p["tp2"]["w"]
    wt1 = padh(wtp2[:_D].reshape(_D, _H, _D))
    wt2 = padh(jnp.pad(wtp2[_D:_D + 9], ((0, 7), (0, 0))).reshape(16, _H, _D))
    wt3 = padh(wtp2[_D + 9:].reshape(_DLEN, _H, _D))
    bt = jnp.pad(p["tp2"]["b"].reshape(_H, 1, _D), ((0, 0), (0, 0), (0, _DP - _D)))
    bt = bt.at[:, :, _D].set(1.0)
    lewh = jnp.pad(p["le"]["w"].reshape(_H, _D, 64), ((0, 0), (0, _DP - _D), (0, 0)))

    shp = jnp.pad(edge_sh, ((0, 0), (0, 7)))

    gath = _sc_gather(node_pad, idx_all)
    alpha8, gmax = _tc_alpha(ele, p)
    p8 = _tc_p(alpha8, gmax)
    heavy_weights = [
        wp1, wp2, wp3, bp, wt1, wt2, wt3, bt, lewh, row(p["le"]["b"]),
        p["s1"]["w"], row(p["s1"]["b"]), row(p["s1g"]), row(p["s1b"]),
        p["s2"]["w"], row(p["s2"]["b"]), row(p["s2g"]), row(p["s2b"]),
        p["s3"]["w"], row(p["s3"]["b"]),
    ]
    pv, esc = _tc_edges(gath, shp, ele, alpha8, p8, heavy_weights)
    zeros128 = jnp.zeros((_NP // _NS, 128), _F32)
    nf2 = _sc_node_scatter(pv, dst, zeros128)
    wl = p["lin"]["w"]
    wl1h = jnp.pad(wl[:_H * _D].reshape(_H, _D, _D), ((0, 0), (0, _DP - _D), (0, 0)))
    node_out = _tc_nodes(nf2, node_in, wl1h, wl[_H * _D:], row(p["lin"]["b"]))
    return node_out, esc


# speedup vs baseline: 7.4081x; 1.2556x over previous
import functools

import jax
import jax.numpy as jnp
from jax import lax
from jax.experimental import pallas as pl
from jax.experimental.pallas import tpu as pltpu
from jax.experimental.pallas import tpu_sc as plsc

_N = 10000
_NP = 10240
_E = 160000
_D = 480
_DP = 512
_H = 4
_DLEN = 128

_NC = 2
_NS = 16
_GB = 80

_F32 = jnp.float32


def _ln(x, g, b):
    mu = jnp.mean(x, axis=-1, keepdims=True)
    var = jnp.mean((x - mu) ** 2, axis=-1, keepdims=True)
    return g * (x - mu) * lax.rsqrt(var + 1e-6) + b


def _silu(x):
    return x * jax.nn.sigmoid(x)


def _sc_gather(table, idx2d):
    mesh = plsc.VectorSubcoreMesh(core_axis_name="c", subcore_axis_name="s")
    blocks = ((2 * _E) // (_NC * _NS)) // _GB
    last = blocks - 1

    @functools.partial(
        pl.kernel,
        mesh=mesh,
        out_type=jax.ShapeDtypeStruct((2 * _E, _DP), _F32),
        scratch_types=[
            pltpu.VMEM((128, _GB), jnp.int32),
            pltpu.VMEM((_GB, _DP), _F32),
            pltpu.VMEM((_GB, _DP), _F32),
            pltpu.SemaphoreType.DMA,
            pltpu.SemaphoreType.DMA,
            pltpu.SemaphoreType.DMA,
            pltpu.SemaphoreType.DMA,
        ],
    )
    def gather_k(tbl_hbm, idx_hbm, out_hbm, idx2, b0, b1, g0, g1, w0, w1):
        c = lax.axis_index("c")
        s = lax.axis_index("s")
        wid = s * _NC + c
        base = wid * blocks * _GB
        pltpu.sync_copy(idx_hbm.at[pl.ds(wid * 128, 128)], idx2)

        def g_start(blk, buf, sem):
            pltpu.async_copy(tbl_hbm.at[idx2.at[blk]], buf, sem)

        def g_wait(blk, buf, sem):
            pltpu.make_async_copy(tbl_hbm.at[idx2.at[blk]], buf, sem).wait()

        def w_start(blk, buf, sem):
            pltpu.async_copy(buf, out_hbm.at[pl.ds(base + blk * _GB, _GB)], sem)

        def w_wait(blk, buf, sem):
            pltpu.make_async_copy(
                buf, out_hbm.at[pl.ds(base + blk * _GB, _GB)], sem).wait()

        g_start(0, b0, g0)
        g_start(1, b1, g1)

        def rnd(r, carry):
            blk0 = 2 * r
            blk1 = 2 * r + 1
            g_wait(blk0, b0, g0)
            w_start(blk0, b0, w0)
            g_wait(blk1, b1, g1)
            w_start(blk1, b1, w1)
            w_wait(blk0, b0, w0)
            g_start(jnp.minimum(blk0 + 2, last), b0, g0)
            w_wait(blk1, b1, w1)
            g_start(jnp.minimum(blk1 + 2, last), b1, g1)
            return carry

        lax.fori_loop(0, last // 2, rnd, 0)
        g_wait(last, b0, g0)
        pltpu.sync_copy(b0, out_hbm.at[pl.ds(base + last * _GB, _GB)])
        g_wait(last, b1, g1)

    return gather_k(table, idx2d)


def _tc_alpha(ele, p):
    be = 2000
    grid = _E // be

    def body(ele_ref, a1w, a1b, a1g, a1bb, a2w, a2b, a2g, a2bb, a3w, a3b,
             alpha_ref, g_ref):
        a = jnp.dot(ele_ref[...], a1w[...], preferred_element_type=_F32) + a1b[...]
        a = _silu(_ln(a, a1g[...], a1bb[...]))
        a = jnp.dot(a, a2w[...], preferred_element_type=_F32) + a2b[...]
        a = _silu(_ln(a, a2g[...], a2bb[...]))
        al8 = jnp.dot(a, a3w[...], preferred_element_type=_F32) + a3b[...]
        alpha_ref[...] = al8
        bm = jnp.max(al8[:, :_H])
        prev = jnp.where(pl.program_id(0) == 0, -3.0e38, g_ref[0, 0])
        g_ref[0, 0] = jnp.maximum(prev, bm)

    row = lambda v: v.reshape(1, -1)
    a3w8 = jnp.pad(p["a3"]["w"], ((0, 0), (0, 8 - _H)))
    a3b8 = row(jnp.pad(p["a3"]["b"], (0, 8 - _H)))
    weights = [
        p["a1"]["w"], row(p["a1"]["b"]), row(p["a1g"]), row(p["a1b"]),
        p["a2"]["w"], row(p["a2"]["b"]), row(p["a2g"]), row(p["a2b"]),
        a3w8, a3b8,
    ]
    in_specs = [pl.BlockSpec((be, _DLEN), lambda i: (i, 0))] + [
        pl.BlockSpec(w.shape, lambda i, nd=w.ndim: (0,) * nd) for w in weights
    ]
    alpha8, gmax = pl.pallas_call(
        body,
        grid=(grid,),
        in_specs=in_specs,
        out_specs=[
            pl.BlockSpec((be, 8), lambda i: (i, 0)),
            pl.BlockSpec((1, 1), lambda i: (0, 0), memory_space=pltpu.SMEM),
        ],
        out_shape=[
            jax.ShapeDtypeStruct((_E, 8), _F32),
            jax.ShapeDtypeStruct((1, 1), _F32),
        ],
    )(ele, *weights)
    return alpha8, gmax


def _tc_p(alpha8, gmax):
    be = 4000
    grid = _E // be

    def body(al_ref, g_ref, p_ref):
        p_ref[...] = jnp.exp(al_ref[...] - g_ref[0, 0])

    return pl.pallas_call(
        body,
        grid=(grid,),
        in_specs=[
            pl.BlockSpec((be, 8), lambda i: (i, 0)),
            pl.BlockSpec((1, 1), lambda i: (0, 0), memory_space=pltpu.SMEM),
        ],
        out_specs=pl.BlockSpec((be, 8), lambda i: (i, 0)),
        out_shape=jax.ShapeDtypeStruct((_E, 8), _F32),
    )(alpha8, gmax)


def _tc_edges(xsxd, shp, ele, alpha8, p8, weights):
    be = 640
    grid = _E // be

    def body(xs_ref, xd_ref, shp_ref, ele_ref, al_ref, p_ref,
             wp1_r, wp2_r, wp3_r, bp_r, wt1_r, wt2_r, wt3_r, bt_r,
             lew_r, leb_r, s1w_r, s1b_r, s1g_r, s1bb_r,
             s2w_r, s2b_r, s2g_r, s2bb_r, s3w_r, s3b_r,
             pv_ref, esc_ref):
        xs = xs_ref[...]
        xd = xd_ref[...]
        ele_b = ele_ref[...]
        msg = (jnp.dot(xs, wp1_r[...], preferred_element_type=_F32)
               + jnp.dot(xd, wp2_r[...], preferred_element_type=_F32)
               + jnp.dot(ele_b, wp3_r[...], preferred_element_type=_F32)
               + bp_r[...])
        shp_b = shp_ref[...]
        al = al_ref[...]
        pw = p_ref[...]
        es = jnp.zeros((be, 64), _F32) + leb_r[...]
        for h in range(_H):
            vh = (jnp.dot(msg, wt1_r[h], preferred_element_type=_F32)
                  + jnp.dot(shp_b, wt2_r[h], preferred_element_type=_F32)
                  + jnp.dot(ele_b, wt3_r[h], preferred_element_type=_F32)
                  + bt_r[h])
            eo = vh * al[:, h:h + 1]
            es = es + jnp.dot(eo, lew_r[h], preferred_element_type=_F32)
            pv_ref[:, h * _DP:(h + 1) * _DP] = vh * pw[:, h:h + 1]
        es = _silu(_ln(jnp.dot(es, s1w_r[...], preferred_element_type=_F32)
                       + s1b_r[...], s1g_r[...], s1bb_r[...]))
        es = _silu(_ln(jnp.dot(es, s2w_r[...], preferred_element_type=_F32)
                       + s2b_r[...], s2g_r[...], s2bb_r[...]))
        esc_ref[...] = jnp.dot(es, s3w_r[...], preferred_element_type=_F32) + s3b_r[...]

    in_specs = [
        pl.BlockSpec((be, _DP), lambda i: (i, 0)),
        pl.BlockSpec((be, _DP), lambda i, g=grid: (i + g, 0)),
        pl.BlockSpec((be, 16), lambda i: (i, 0)),
        pl.BlockSpec((be, _DLEN), lambda i: (i, 0)),
        pl.BlockSpec((be, 8), lambda i: (i, 0)),
        pl.BlockSpec((be, 8), lambda i: (i, 0)),
    ] + [pl.BlockSpec(w.shape, lambda i, nd=w.ndim: (0,) * nd) for w in weights]

    pv, esc = pl.pallas_call(
        body,
        grid=(grid,),
        in_specs=in_specs,
        out_specs=[
            pl.BlockSpec((be, _H * _DP), lambda i: (i, 0)),
            pl.BlockSpec((be, 32), lambda i: (i, 0)),
        ],
        out_shape=[
            jax.ShapeDtypeStruct((_E, _H * _DP), _F32),
            jax.ShapeDtypeStruct((_E, 32), _F32),
        ],
    )(xsxd, xsxd, shp, ele, alpha8, p8, *weights)
    return pv, esc


def _sc_node_scatter(pv2, dst, zeros128):
    mesh = plsc.VectorSubcoreMesh(core_axis_name="c", subcore_axis_name="s")
    e_per_t = _E // _NS
    iters = e_per_t // _GB
    cw = 128
    chunks_per_sc = (_H * _DP) // cw // _NC
    rows_per_t = _NP // _NS

    nbuf = 2
    last = iters - 1

    @functools.partial(
        pl.kernel,
        mesh=mesh,
        out_type=jax.ShapeDtypeStruct((_NP, _H * _DP), _F32),
        scratch_types=[
            pltpu.VMEM_SHARED((_NP, cw), _F32),
            pltpu.VMEM((128, _GB), jnp.int32),
            [pltpu.VMEM((_GB, cw), _F32) for _ in range(nbuf)],
            [pltpu.SemaphoreType.DMA for _ in range(nbuf)],
            [pltpu.SemaphoreType.DMA for _ in range(nbuf)],
        ],
    )
    def nf_k(pv_hbm, dst_hbm, z_hbm, out_hbm, acc, idx2, bufs, lsems, ssems):
        c = lax.axis_index("c")
        s = lax.axis_index("s")
        r0 = s * rows_per_t
        pltpu.sync_copy(dst_hbm.at[pl.ds(s * 128, 128)], idx2)

        def ld_start(blk, co, b, sem):
            pltpu.async_copy(
                pv_hbm.at[pl.ds(s * e_per_t + blk * _GB, _GB), pl.ds(co, cw)],
                b, sem)

        def ld_wait(blk, co, b, sem):
            pltpu.make_async_copy(
                pv_hbm.at[pl.ds(s * e_per_t + blk * _GB, _GB), pl.ds(co, cw)],
                b, sem).wait()

        def sc_start(blk, b, sem):
            pltpu.async_copy(b, acc.at[idx2.at[blk]], sem, add=True)

        def sc_wait(blk, b, sem):
            pltpu.make_async_copy(b, acc.at[idx2.at[blk]], sem).wait()

        for j in range(chunks_per_sc):
            co = (c * chunks_per_sc + j) * cw
            for b in range(nbuf):
                ld_start(b, co, bufs[b], lsems[b])
            pltpu.sync_copy(z_hbm, acc.at[pl.ds(r0, rows_per_t)])
            plsc.subcore_barrier()

            def rnd(r, carry):
                base_blk = nbuf * r
                for b in range(nbuf):
                    ld_wait(base_blk + b, co, bufs[b], lsems[b])
                    sc_start(base_blk + b, bufs[b], ssems[b])
                for b in range(nbuf):
                    sc_wait(base_blk + b, bufs[b], ssems[b])
                    ld_start(jnp.minimum(base_blk + nbuf + b, last), co,
                             bufs[b], lsems[b])
                return carry

            lax.fori_loop(0, last // nbuf, rnd, 0)
            for b in range(nbuf):
                ld_wait(last, co, bufs[b], lsems[b])
            pltpu.sync_copy(bufs[0], acc.at[idx2.at[last]], add=True)
            plsc.subcore_barrier()
            pltpu.sync_copy(acc.at[pl.ds(r0, rows_per_t)],
                            out_hbm.at[pl.ds(r0, rows_per_t), pl.ds(co, cw)])
            plsc.subcore_barrier()

    return nf_k(pv2, dst, zeros128)


def _tc_nodes(nf2, node_in, wl1h, wl2, bl):
    bn = 400
    grid = _N // bn

    def body(nf_ref, x_ref, wl1_r, wl2_r, bl_r, out_ref):
        acc = jnp.dot(x_ref[...], wl2_r[...], preferred_element_type=_F32) + bl_r[...]
        for h in range(_H):
            nh = nf_ref[:, h * _DP:(h + 1) * _DP]
            sden = nh[:, _D:_D + 1]
            inv = jnp.where(sden > 0, 1.0 / sden, 0.0)
            acc = acc + jnp.dot(nh * inv, wl1_r[h], preferred_element_type=_F32)
        out_ref[...] = acc

    weights = [wl1h, wl2, bl]
    in_specs = [
        pl.BlockSpec((bn, _H * _DP), lambda i: (i, 0)),
        pl.BlockSpec((bn, _D), lambda i: (i, 0)),
    ] + [pl.BlockSpec(w.shape, lambda i, nd=w.ndim: (0,) * nd) for w in weights]

    return pl.pallas_call(
        body,
        grid=(grid,),
        in_specs=in_specs,
        out_specs=pl.BlockSpec((bn, _D), lambda i: (i, 0)),
        out_shape=jax.ShapeDtypeStruct((_N, _D), _F32),
    )(nf2, node_in, *weights)


def kernel(node_in, node_embed, edge_sh, edge_length_embedding, edge_src,
           edge_dst, batch, params):
    p = params
    ele = edge_length_embedding
    src = edge_src.astype(jnp.int32)
    dst = edge_dst.astype(jnp.int32)
    idx_all = jnp.concatenate([src, dst])
    idx2d = jnp.pad(idx_all.reshape(32, 125, _GB),
                    ((0, 0), (0, 3), (0, 0))).reshape(32 * 128, _GB)
    dst2d = jnp.pad(dst.reshape(_NS, 125, _GB),
                    ((0, 0), (0, 3), (0, 0))).reshape(_NS * 128, _GB)
    row = lambda v: v.reshape(1, -1)

    node_pad = jnp.pad(node_in, ((0, 0), (0, _DP - _D)))
    wpre = p["pre"]["w"]
    padk = lambda w: jnp.pad(w, ((0, _DP - _D), (0, 0)))
    wp1, wp2, wp3 = padk(wpre[:_D]), padk(wpre[_D:2 * _D]), wpre[2 * _D:]
    bp = row(p["pre"]["b"])

    def padh(w3):
        return jnp.pad(w3.transpose(1, 0, 2), ((0, 0), (0, 0), (0, _DP - _D)))

    wtp2 = p["tp2"]["w"]
    wt1 = padh(wtp2[:_D].reshape(_D, _H, _D))
    wt2 = padh(jnp.pad(wtp2[_D:_D + 9], ((0, 7), (0, 0))).reshape(16, _H, _D))
    wt3 = padh(wtp2[_D + 9:].reshape(_DLEN, _H, _D))
    bt = jnp.pad(p["tp2"]["b"].reshape(_H, 1, _D), ((0, 0), (0, 0), (0, _DP - _D)))
    bt = bt.at[:, :, _D].set(1.0)
    lewh = jnp.pad(p["le"]["w"].reshape(_H, _D, 64), ((0, 0), (0, _DP - _D), (0, 0)))

    shp = jnp.pad(edge_sh, ((0, 0), (0, 7)))

    gath = _sc_gather(node_pad, idx2d)
    alpha8, gmax = _tc_alpha(ele, p)
    p8 = _tc_p(alpha8, gmax)
    heavy_weights = [
        wp1, wp2, wp3, bp, wt1, wt2, wt3, bt, lewh, row(p["le"]["b"]),
        p["s1"]["w"], row(p["s1"]["b"]), row(p["s1g"]), row(p["s1b"]),
        p["s2"]["w"], row(p["s2"]["b"]), row(p["s2g"]), row(p["s2b"]),
        p["s3"]["w"], row(p["s3"]["b"]),
    ]
    pv, esc = _tc_edges(gath, shp, ele, alpha8, p8, heavy_weights)
    zeros128 = jnp.zeros((_NP // _NS, 128), _F32)
    nf2 = _sc_node_scatter(pv, dst2d, zeros128)
    wl = p["lin"]["w"]
    wl1h = jnp.pad(wl[:_H * _D].reshape(_H, _D, _D), ((0, 0), (0, _DP - _D), (0, 0)))
    node_out = _tc_nodes(nf2, node_in, wl1h, wl[_H * _D:], row(p["lin"]["b"]))
    return node_out, esc


# speedup vs baseline: 8.1208x; 1.0962x over previous
import functools

import jax
import jax.numpy as jnp
from jax import lax
from jax.experimental import pallas as pl
from jax.experimental.pallas import tpu as pltpu
from jax.experimental.pallas import tpu_sc as plsc

_N = 10000
_NP = 10240
_E = 160000
_D = 480
_DP = 512
_H = 4
_DLEN = 128

_NC = 2
_NS = 16
_GB = 80

_F32 = jnp.float32


def _ln(x, g, b):
    mu = jnp.mean(x, axis=-1, keepdims=True)
    var = jnp.mean((x - mu) ** 2, axis=-1, keepdims=True)
    return g * (x - mu) * lax.rsqrt(var + 1e-6) + b


def _silu(x):
    return x * jax.nn.sigmoid(x)


def _sc_gather(table, idx2d):
    mesh = plsc.VectorSubcoreMesh(core_axis_name="c", subcore_axis_name="s")
    blocks = ((2 * _E) // (_NC * _NS)) // _GB
    last = blocks - 1

    @functools.partial(
        pl.kernel,
        mesh=mesh,
        out_type=jax.ShapeDtypeStruct((2 * _E, _DP), _F32),
        scratch_types=[
            pltpu.VMEM((128, _GB), jnp.int32),
            pltpu.VMEM((_GB, _DP), _F32),
            pltpu.VMEM((_GB, _DP), _F32),
            pltpu.SemaphoreType.DMA,
            pltpu.SemaphoreType.DMA,
            pltpu.SemaphoreType.DMA,
            pltpu.SemaphoreType.DMA,
        ],
    )
    def gather_k(tbl_hbm, idx_hbm, out_hbm, idx2, b0, b1, g0, g1, w0, w1):
        c = lax.axis_index("c")
        s = lax.axis_index("s")
        wid = s * _NC + c
        base = wid * blocks * _GB
        pltpu.sync_copy(idx_hbm.at[pl.ds(wid * 128, 128)], idx2)

        def g_start(blk, buf, sem):
            pltpu.async_copy(tbl_hbm.at[idx2.at[blk]], buf, sem)

        def g_wait(blk, buf, sem):
            pltpu.make_async_copy(tbl_hbm.at[idx2.at[blk]], buf, sem).wait()

        def w_start(blk, buf, sem):
            pltpu.async_copy(buf, out_hbm.at[pl.ds(base + blk * _GB, _GB)], sem)

        def w_wait(blk, buf, sem):
            pltpu.make_async_copy(
                buf, out_hbm.at[pl.ds(base + blk * _GB, _GB)], sem).wait()

        g_start(0, b0, g0)
        g_start(1, b1, g1)

        def rnd(r, carry):
            blk0 = 2 * r
            blk1 = 2 * r + 1
            g_wait(blk0, b0, g0)
            w_start(blk0, b0, w0)
            g_wait(blk1, b1, g1)
            w_start(blk1, b1, w1)
            w_wait(blk0, b0, w0)
            g_start(jnp.minimum(blk0 + 2, last), b0, g0)
            w_wait(blk1, b1, w1)
            g_start(jnp.minimum(blk1 + 2, last), b1, g1)
            return carry

        lax.fori_loop(0, last // 2, rnd, 0)
        g_wait(last, b0, g0)
        pltpu.sync_copy(b0, out_hbm.at[pl.ds(base + last * _GB, _GB)])
        g_wait(last, b1, g1)

    return gather_k(table, idx2d)


def _tc_alpha(ele, p):
    be = 2000
    grid = _E // be

    def body(ele_ref, a1w, a1b, a1g, a1bb, a2w, a2b, a2g, a2bb, a3w, a3b,
             alpha_ref, g_ref):
        a = jnp.dot(ele_ref[...], a1w[...], preferred_element_type=_F32) + a1b[...]
        a = _silu(_ln(a, a1g[...], a1bb[...]))
        a = jnp.dot(a, a2w[...], preferred_element_type=_F32) + a2b[...]
        a = _silu(_ln(a, a2g[...], a2bb[...]))
        al8 = jnp.dot(a, a3w[...], preferred_element_type=_F32) + a3b[...]
        alpha_ref[...] = al8
        bm = jnp.max(al8[:, :_H])
        prev = jnp.where(pl.program_id(0) == 0, -3.0e38, g_ref[0, 0])
        g_ref[0, 0] = jnp.maximum(prev, bm)

    row = lambda v: v.reshape(1, -1)
    a3w8 = jnp.pad(p["a3"]["w"], ((0, 0), (0, 8 - _H)))
    a3b8 = row(jnp.pad(p["a3"]["b"], (0, 8 - _H)))
    weights = [
        p["a1"]["w"], row(p["a1"]["b"]), row(p["a1g"]), row(p["a1b"]),
        p["a2"]["w"], row(p["a2"]["b"]), row(p["a2g"]), row(p["a2b"]),
        a3w8, a3b8,
    ]
    in_specs = [pl.BlockSpec((be, _DLEN), lambda i: (i, 0))] + [
        pl.BlockSpec(w.shape, lambda i, nd=w.ndim: (0,) * nd) for w in weights
    ]
    alpha8, gmax = pl.pallas_call(
        body,
        grid=(grid,),
        in_specs=in_specs,
        out_specs=[
            pl.BlockSpec((be, 8), lambda i: (i, 0)),
            pl.BlockSpec((1, 1), lambda i: (0, 0), memory_space=pltpu.SMEM),
        ],
        out_shape=[
            jax.ShapeDtypeStruct((_E, 8), _F32),
            jax.ShapeDtypeStruct((1, 1), _F32),
        ],
    )(ele, *weights)
    return alpha8, gmax


def _tc_edges(xsxd, shp, ele, alpha8, gmax, weights):
    be = 640
    grid = _E // be

    def body(xs_ref, xd_ref, shp_ref, ele_ref, al_ref, g_ref,
             wp1_r, wp2_r, wp3_r, bp_r, wt1_r, wt2_r, wt3_r, bt_r,
             lew_r, leb_r, s1w_r, s1b_r, s1g_r, s1bb_r,
             s2w_r, s2b_r, s2g_r, s2bb_r, s3w_r, s3b_r,
             pv_ref, esc_ref):
        xs = xs_ref[...]
        xd = xd_ref[...]
        ele_b = ele_ref[...]
        msg = (jnp.dot(xs, wp1_r[...], preferred_element_type=_F32)
               + jnp.dot(xd, wp2_r[...], preferred_element_type=_F32)
               + jnp.dot(ele_b, wp3_r[...], preferred_element_type=_F32)
               + bp_r[...])
        shp_b = shp_ref[...]
        al = al_ref[...]
        pw = jnp.exp(al - g_ref[0, 0])
        es = jnp.zeros((be, 64), _F32) + leb_r[...]
        for h in range(_H):
            vh = (jnp.dot(msg, wt1_r[h], preferred_element_type=_F32)
                  + jnp.dot(shp_b, wt2_r[h], preferred_element_type=_F32)
                  + jnp.dot(ele_b, wt3_r[h], preferred_element_type=_F32)
                  + bt_r[h])
            eo = vh * al[:, h:h + 1]
            es = es + jnp.dot(eo, lew_r[h], preferred_element_type=_F32)
            pv_ref[:, h * _DP:(h + 1) * _DP] = vh * pw[:, h:h + 1]
        es = _silu(_ln(jnp.dot(es, s1w_r[...], preferred_element_type=_F32)
                       + s1b_r[...], s1g_r[...], s1bb_r[...]))
        es = _silu(_ln(jnp.dot(es, s2w_r[...], preferred_element_type=_F32)
                       + s2b_r[...], s2g_r[...], s2bb_r[...]))
        esc_ref[...] = jnp.dot(es, s3w_r[...], preferred_element_type=_F32) + s3b_r[...]

    in_specs = [
        pl.BlockSpec((be, _DP), lambda i: (i, 0)),
        pl.BlockSpec((be, _DP), lambda i, g=grid: (i + g, 0)),
        pl.BlockSpec((be, 16), lambda i: (i, 0)),
        pl.BlockSpec((be, _DLEN), lambda i: (i, 0)),
        pl.BlockSpec((be, 8), lambda i: (i, 0)),
        pl.BlockSpec((1, 1), lambda i: (0, 0), memory_space=pltpu.SMEM),
    ] + [pl.BlockSpec(w.shape, lambda i, nd=w.ndim: (0,) * nd) for w in weights]

    pv, esc = pl.pallas_call(
        body,
        grid=(grid,),
        in_specs=in_specs,
        out_specs=[
            pl.BlockSpec((be, _H * _DP), lambda i: (i, 0)),
            pl.BlockSpec((be, 32), lambda i: (i, 0)),
        ],
        out_shape=[
            jax.ShapeDtypeStruct((_E, _H * _DP), _F32),
            jax.ShapeDtypeStruct((_E, 32), _F32),
        ],
    )(xsxd, xsxd, shp, ele, alpha8, gmax, *weights)
    return pv, esc


def _sc_node_scatter(pv2, dst, zeros128):
    mesh = plsc.VectorSubcoreMesh(core_axis_name="c", subcore_axis_name="s")
    e_per_t = _E // _NS
    iters = e_per_t // _GB
    cw = 128
    chunks_per_sc = (_H * _DP) // cw // _NC
    rows_per_t = _NP // _NS

    nbuf = 3
    last = iters - 1

    @functools.partial(
        pl.kernel,
        mesh=mesh,
        out_type=jax.ShapeDtypeStruct((_NP, _H * _DP), _F32),
        scratch_types=[
            pltpu.VMEM_SHARED((_NP, cw), _F32),
            pltpu.VMEM((128, _GB), jnp.int32),
            [pltpu.VMEM((_GB, cw), _F32) for _ in range(nbuf)],
            [pltpu.SemaphoreType.DMA for _ in range(nbuf)],
            [pltpu.SemaphoreType.DMA for _ in range(nbuf)],
        ],
    )
    def nf_k(pv_hbm, dst_hbm, z_hbm, out_hbm, acc, idx2, bufs, lsems, ssems):
        c = lax.axis_index("c")
        s = lax.axis_index("s")
        r0 = s * rows_per_t
        pltpu.sync_copy(dst_hbm.at[pl.ds(s * 128, 128)], idx2)

        def ld_start(blk, co, b, sem):
            pltpu.async_copy(
                pv_hbm.at[pl.ds(s * e_per_t + blk * _GB, _GB), pl.ds(co, cw)],
                b, sem)

        def ld_wait(blk, co, b, sem):
            pltpu.make_async_copy(
                pv_hbm.at[pl.ds(s * e_per_t + blk * _GB, _GB), pl.ds(co, cw)],
                b, sem).wait()

        def sc_start(blk, b, sem):
            pltpu.async_copy(b, acc.at[idx2.at[blk]], sem, add=True)

        def sc_wait(blk, b, sem):
            pltpu.make_async_copy(b, acc.at[idx2.at[blk]], sem).wait()

        for j in range(chunks_per_sc):
            co = (c * chunks_per_sc + j) * cw
            for b in range(nbuf):
                ld_start(b, co, bufs[b], lsems[b])
            pltpu.sync_copy(z_hbm, acc.at[pl.ds(r0, rows_per_t)])
            plsc.subcore_barrier()

            def rnd(r, carry):
                base_blk = nbuf * r
                for b in range(nbuf):
                    ld_wait(base_blk + b, co, bufs[b], lsems[b])
                    sc_start(base_blk + b, bufs[b], ssems[b])
                for b in range(nbuf):
                    sc_wait(base_blk + b, bufs[b], ssems[b])
                    ld_start(jnp.minimum(base_blk + nbuf + b, last), co,
                             bufs[b], lsems[b])
                return carry

            full_rounds = iters // nbuf
            rem = iters - full_rounds * nbuf
            lax.fori_loop(0, full_rounds, rnd, 0)
            for b in range(nbuf):
                ld_wait(min(full_rounds * nbuf + b, last), co,
                        bufs[b], lsems[b])
            for b in range(rem):
                pltpu.sync_copy(bufs[b],
                                acc.at[idx2.at[full_rounds * nbuf + b]],
                                add=True)
            plsc.subcore_barrier()
            pltpu.sync_copy(acc.at[pl.ds(r0, rows_per_t)],
                            out_hbm.at[pl.ds(r0, rows_per_t), pl.ds(co, cw)])
            plsc.subcore_barrier()

    return nf_k(pv2, dst, zeros128)


def _tc_nodes(nf2, node_in, wl1h, wl2, bl):
    bn = 400
    grid = _N // bn

    def body(nf_ref, x_ref, wl1_r, wl2_r, bl_r, out_ref):
        acc = jnp.dot(x_ref[...], wl2_r[...], preferred_element_type=_F32) + bl_r[...]
        for h in range(_H):
            nh = nf_ref[:, h * _DP:(h + 1) * _DP]
            sden = nh[:, _D:_D + 1]
            inv = jnp.where(sden > 0, 1.0 / sden, 0.0)
            acc = acc + jnp.dot(nh * inv, wl1_r[h], preferred_element_type=_F32)
        out_ref[...] = acc

    weights = [wl1h, wl2, bl]
    in_specs = [
        pl.BlockSpec((bn, _H * _DP), lambda i: (i, 0)),
        pl.BlockSpec((bn, _D), lambda i: (i, 0)),
    ] + [pl.BlockSpec(w.shape, lambda i, nd=w.ndim: (0,) * nd) for w in weights]

    return pl.pallas_call(
        body,
        grid=(grid,),
        in_specs=in_specs,
        out_specs=pl.BlockSpec((bn, _D), lambda i: (i, 0)),
        out_shape=jax.ShapeDtypeStruct((_N, _D), _F32),
    )(nf2, node_in, *weights)


def kernel(node_in, node_embed, edge_sh, edge_length_embedding, edge_src,
           edge_dst, batch, params):
    p = params
    ele = edge_length_embedding
    src = edge_src.astype(jnp.int32)
    dst = edge_dst.astype(jnp.int32)
    idx_all = jnp.concatenate([src, dst])
    idx2d = jnp.pad(idx_all.reshape(32, 125, _GB),
                    ((0, 0), (0, 3), (0, 0))).reshape(32 * 128, _GB)
    dst2d = jnp.pad(dst.reshape(_NS, 125, _GB),
                    ((0, 0), (0, 3), (0, 0))).reshape(_NS * 128, _GB)
    row = lambda v: v.reshape(1, -1)

    node_pad = jnp.pad(node_in, ((0, 0), (0, _DP - _D)))
    wpre = p["pre"]["w"]
    padk = lambda w: jnp.pad(w, ((0, _DP - _D), (0, 0)))
    wp1, wp2, wp3 = padk(wpre[:_D]), padk(wpre[_D:2 * _D]), wpre[2 * _D:]
    bp = row(p["pre"]["b"])

    def padh(w3):
        return jnp.pad(w3.transpose(1, 0, 2), ((0, 0), (0, 0), (0, _DP - _D)))

    wtp2 = p["tp2"]["w"]
    wt1 = padh(wtp2[:_D].reshape(_D, _H, _D))
    wt2 = padh(jnp.pad(wtp2[_D:_D + 9], ((0, 7), (0, 0))).reshape(16, _H, _D))
    wt3 = padh(wtp2[_D + 9:].reshape(_DLEN, _H, _D))
    bt = jnp.pad(p["tp2"]["b"].reshape(_H, 1, _D), ((0, 0), (0, 0), (0, _DP - _D)))
    bt = bt.at[:, :, _D].set(1.0)
    lewh = jnp.pad(p["le"]["w"].reshape(_H, _D, 64), ((0, 0), (0, _DP - _D), (0, 0)))

    shp = jnp.pad(edge_sh, ((0, 0), (0, 7)))

    gath = _sc_gather(node_pad, idx2d)
    alpha8, gmax = _tc_alpha(ele, p)
    heavy_weights = [
        wp1, wp2, wp3, bp, wt1, wt2, wt3, bt, lewh, row(p["le"]["b"]),
        p["s1"]["w"], row(p["s1"]["b"]), row(p["s1g"]), row(p["s1b"]),
        p["s2"]["w"], row(p["s2"]["b"]), row(p["s2g"]), row(p["s2b"]),
        p["s3"]["w"], row(p["s3"]["b"]),
    ]
    pv, esc = _tc_edges(gath, shp, ele, alpha8, gmax, heavy_weights)
    zeros128 = jnp.zeros((_NP // _NS, 128), _F32)
    nf2 = _sc_node_scatter(pv, dst2d, zeros128)
    wl = p["lin"]["w"]
    wl1h = jnp.pad(wl[:_H * _D].reshape(_H, _D, _D), ((0, 0), (0, _DP - _D), (0, 0)))
    node_out = _tc_nodes(nf2, node_in, wl1h, wl[_H * _D:], row(p["lin"]["b"]))
    return node_out, esc


# speedup vs baseline: 9.0308x; 1.1121x over previous
import functools

import jax
import jax.numpy as jnp
from jax import lax
from jax.experimental import pallas as pl
from jax.experimental.pallas import tpu as pltpu
from jax.experimental.pallas import tpu_sc as plsc

_N = 10000
_NP = 10240
_E = 160000
_D = 480
_DP = 512
_H = 4
_DLEN = 128

_NC = 2
_NS = 16
_GB = 80

_F32 = jnp.float32


def _ln(x, g, b):
    mu = jnp.mean(x, axis=-1, keepdims=True)
    var = jnp.mean((x - mu) ** 2, axis=-1, keepdims=True)
    return g * (x - mu) * lax.rsqrt(var + 1e-6) + b


def _silu(x):
    return x * jax.nn.sigmoid(x)


def _sc_gather(table, idx2d):
    mesh = plsc.VectorSubcoreMesh(core_axis_name="c", subcore_axis_name="s")
    blocks = ((2 * _E) // (_NC * _NS)) // _GB
    last = blocks - 1

    @functools.partial(
        pl.kernel,
        mesh=mesh,
        out_type=jax.ShapeDtypeStruct((2 * _E, _DP), _F32),
        scratch_types=[
            pltpu.VMEM((128, _GB), jnp.int32),
            pltpu.VMEM((_GB, _DP), _F32),
            pltpu.VMEM((_GB, _DP), _F32),
            pltpu.SemaphoreType.DMA,
            pltpu.SemaphoreType.DMA,
            pltpu.SemaphoreType.DMA,
            pltpu.SemaphoreType.DMA,
        ],
    )
    def gather_k(tbl_hbm, idx_hbm, out_hbm, idx2, b0, b1, g0, g1, w0, w1):
        c = lax.axis_index("c")
        s = lax.axis_index("s")
        wid = s * _NC + c
        base = wid * blocks * _GB
        pltpu.sync_copy(idx_hbm.at[pl.ds(wid * 128, 128)], idx2)

        def g_start(blk, buf, sem):
            pltpu.async_copy(tbl_hbm.at[idx2.at[blk]], buf, sem)

        def g_wait(blk, buf, sem):
            pltpu.make_async_copy(tbl_hbm.at[idx2.at[blk]], buf, sem).wait()

        def w_start(blk, buf, sem):
            pltpu.async_copy(buf, out_hbm.at[pl.ds(base + blk * _GB, _GB)], sem)

        def w_wait(blk, buf, sem):
            pltpu.make_async_copy(
                buf, out_hbm.at[pl.ds(base + blk * _GB, _GB)], sem).wait()

        g_start(0, b0, g0)
        g_start(1, b1, g1)

        def rnd(r, carry):
            blk0 = 2 * r
            blk1 = 2 * r + 1
            g_wait(blk0, b0, g0)
            w_start(blk0, b0, w0)
            g_wait(blk1, b1, g1)
            w_start(blk1, b1, w1)
            w_wait(blk0, b0, w0)
            g_start(jnp.minimum(blk0 + 2, last), b0, g0)
            w_wait(blk1, b1, w1)
            g_start(jnp.minimum(blk1 + 2, last), b1, g1)
            return carry

        lax.fori_loop(0, last // 2, rnd, 0)
        g_wait(last, b0, g0)
        pltpu.sync_copy(b0, out_hbm.at[pl.ds(base + last * _GB, _GB)])
        g_wait(last, b1, g1)

    return gather_k(table, idx2d)


def _tc_alpha(ele, p):
    be = 2000
    grid = _E // be

    def body(ele_ref, a1w, a1b, a1g, a1bb, a2w, a2b, a2g, a2bb, a3w, a3b,
             alpha_ref, g_ref):
        a = jnp.dot(ele_ref[...], a1w[...], preferred_element_type=_F32) + a1b[...]
        a = _silu(_ln(a, a1g[...], a1bb[...]))
        a = jnp.dot(a, a2w[...], preferred_element_type=_F32) + a2b[...]
        a = _silu(_ln(a, a2g[...], a2bb[...]))
        al8 = jnp.dot(a, a3w[...], preferred_element_type=_F32) + a3b[...]
        alpha_ref[...] = al8
        bm = jnp.max(al8[:, :_H])
        prev = jnp.where(pl.program_id(0) == 0, -3.0e38, g_ref[0, 0])
        g_ref[0, 0] = jnp.maximum(prev, bm)

    row = lambda v: v.reshape(1, -1)
    a3w8 = jnp.pad(p["a3"]["w"], ((0, 0), (0, 8 - _H)))
    a3b8 = row(jnp.pad(p["a3"]["b"], (0, 8 - _H)))
    weights = [
        p["a1"]["w"], row(p["a1"]["b"]), row(p["a1g"]), row(p["a1b"]),
        p["a2"]["w"], row(p["a2"]["b"]), row(p["a2g"]), row(p["a2b"]),
        a3w8, a3b8,
    ]
    in_specs = [pl.BlockSpec((be, _DLEN), lambda i: (i, 0))] + [
        pl.BlockSpec(w.shape, lambda i, nd=w.ndim: (0,) * nd) for w in weights
    ]
    alpha8, gmax = pl.pallas_call(
        body,
        grid=(grid,),
        in_specs=in_specs,
        out_specs=[
            pl.BlockSpec((be, 8), lambda i: (i, 0)),
            pl.BlockSpec((1, 1), lambda i: (0, 0), memory_space=pltpu.SMEM),
        ],
        out_shape=[
            jax.ShapeDtypeStruct((_E, 8), _F32),
            jax.ShapeDtypeStruct((1, 1), _F32),
        ],
    )(ele, *weights)
    return alpha8, gmax


def _tc_edges(xsxd, shp, ele, alpha8, gmax, weights, e_start, e_count):
    be = 640
    grid = e_count // be
    ofs = e_start // be
    xd_ofs = (_E + e_start) // be

    def body(xs_ref, xd_ref, shp_ref, ele_ref, al_ref, g_ref,
             wp1_r, wp2_r, wp3_r, bp_r, wt1_r, wt2_r, wt3_r, bt_r,
             lew_r, leb_r, s1w_r, s1b_r, s1g_r, s1bb_r,
             s2w_r, s2b_r, s2g_r, s2bb_r, s3w_r, s3b_r,
             pv_ref, esc_ref):
        xs = xs_ref[...]
        xd = xd_ref[...]
        ele_b = ele_ref[...]
        msg = (jnp.dot(xs, wp1_r[...], preferred_element_type=_F32)
               + jnp.dot(xd, wp2_r[...], preferred_element_type=_F32)
               + jnp.dot(ele_b, wp3_r[...], preferred_element_type=_F32)
               + bp_r[...])
        shp_b = shp_ref[...]
        al = al_ref[...]
        pw = jnp.exp(al - g_ref[0, 0])
        es = jnp.zeros((be, 64), _F32) + leb_r[...]
        for h in range(_H):
            vh = (jnp.dot(msg, wt1_r[h], preferred_element_type=_F32)
                  + jnp.dot(shp_b, wt2_r[h], preferred_element_type=_F32)
                  + jnp.dot(ele_b, wt3_r[h], preferred_element_type=_F32)
                  + bt_r[h])
            eo = vh * al[:, h:h + 1]
            es = es + jnp.dot(eo, lew_r[h], preferred_element_type=_F32)
            pv_ref[:, h * _DP:(h + 1) * _DP] = vh * pw[:, h:h + 1]
        es = _silu(_ln(jnp.dot(es, s1w_r[...], preferred_element_type=_F32)
                       + s1b_r[...], s1g_r[...], s1bb_r[...]))
        es = _silu(_ln(jnp.dot(es, s2w_r[...], preferred_element_type=_F32)
                       + s2b_r[...], s2g_r[...], s2bb_r[...]))
        esc_ref[...] = jnp.dot(es, s3w_r[...], preferred_element_type=_F32) + s3b_r[...]

    in_specs = [
        pl.BlockSpec((be, _DP), lambda i, o=ofs: (i + o, 0)),
        pl.BlockSpec((be, _DP), lambda i, o=xd_ofs: (i + o, 0)),
        pl.BlockSpec((be, 16), lambda i, o=ofs: (i + o, 0)),
        pl.BlockSpec((be, _DLEN), lambda i, o=ofs: (i + o, 0)),
        pl.BlockSpec((be, 8), lambda i, o=ofs: (i + o, 0)),
        pl.BlockSpec((1, 1), lambda i: (0, 0), memory_space=pltpu.SMEM),
    ] + [pl.BlockSpec(w.shape, lambda i, nd=w.ndim: (0,) * nd) for w in weights]

    pv, esc = pl.pallas_call(
        body,
        grid=(grid,),
        in_specs=in_specs,
        out_specs=[
            pl.BlockSpec((be, _H * _DP), lambda i: (i, 0)),
            pl.BlockSpec((be, 32), lambda i: (i, 0)),
        ],
        out_shape=[
            jax.ShapeDtypeStruct((e_count, _H * _DP), _F32),
            jax.ShapeDtypeStruct((e_count, 32), _F32),
        ],
    )(xsxd, xsxd, shp, ele, alpha8, gmax, *weights)
    return pv, esc


def _sc_node_scatter(pv2, dst2d, zeros128, iters):
    mesh = plsc.VectorSubcoreMesh(core_axis_name="c", subcore_axis_name="s")
    e_per_t = iters * _GB
    cw = 128
    chunks_per_sc = (_H * _DP) // cw // _NC
    rows_per_t = _NP // _NS

    nbuf = 3
    last = iters - 1

    @functools.partial(
        pl.kernel,
        mesh=mesh,
        out_type=jax.ShapeDtypeStruct((_NP, _H * _DP), _F32),
        scratch_types=[
            pltpu.VMEM_SHARED((_NP, cw), _F32),
            pltpu.VMEM((64, _GB), jnp.int32),
            [pltpu.VMEM((_GB, cw), _F32) for _ in range(nbuf)],
            [pltpu.SemaphoreType.DMA for _ in range(nbuf)],
            [pltpu.SemaphoreType.DMA for _ in range(nbuf)],
        ],
    )
    def nf_k(pv_hbm, dst_hbm, z_hbm, out_hbm, acc, idx2, bufs, lsems, ssems):
        c = lax.axis_index("c")
        s = lax.axis_index("s")
        r0 = s * rows_per_t
        pltpu.sync_copy(dst_hbm.at[pl.ds(s * 64, 64)], idx2)

        def ld_start(blk, co, b, sem):
            pltpu.async_copy(
                pv_hbm.at[pl.ds(s * e_per_t + blk * _GB, _GB), pl.ds(co, cw)],
                b, sem)

        def ld_wait(blk, co, b, sem):
            pltpu.make_async_copy(
                pv_hbm.at[pl.ds(s * e_per_t + blk * _GB, _GB), pl.ds(co, cw)],
                b, sem).wait()

        def sc_start(blk, b, sem):
            pltpu.async_copy(b, acc.at[idx2.at[blk]], sem, add=True)

        def sc_wait(blk, b, sem):
            pltpu.make_async_copy(b, acc.at[idx2.at[blk]], sem).wait()

        for j in range(chunks_per_sc):
            co = (c * chunks_per_sc + j) * cw
            for b in range(nbuf):
                ld_start(b, co, bufs[b], lsems[b])
            pltpu.sync_copy(z_hbm, acc.at[pl.ds(r0, rows_per_t)])
            plsc.subcore_barrier()

            def rnd(r, carry):
                base_blk = nbuf * r
                for b in range(nbuf):
                    ld_wait(base_blk + b, co, bufs[b], lsems[b])
                    sc_start(base_blk + b, bufs[b], ssems[b])
                for b in range(nbuf):
                    sc_wait(base_blk + b, bufs[b], ssems[b])
                    ld_start(jnp.minimum(base_blk + nbuf + b, last), co,
                             bufs[b], lsems[b])
                return carry

            full_rounds = iters // nbuf
            rem = iters - full_rounds * nbuf
            lax.fori_loop(0, full_rounds, rnd, 0)
            for b in range(nbuf):
                ld_wait(min(full_rounds * nbuf + b, last), co,
                        bufs[b], lsems[b])
            for b in range(rem):
                pltpu.sync_copy(bufs[b],
                                acc.at[idx2.at[full_rounds * nbuf + b]],
                                add=True)
            plsc.subcore_barrier()
            pltpu.sync_copy(acc.at[pl.ds(r0, rows_per_t)],
                            out_hbm.at[pl.ds(r0, rows_per_t), pl.ds(co, cw)])
            plsc.subcore_barrier()

    return nf_k(pv2, dst2d, zeros128)


def _tc_nodes(nf_a, nf_b, node_in, wl1h, wl2, bl):
    bn = 400
    grid = _N // bn

    def body(nfa_ref, nfb_ref, x_ref, wl1_r, wl2_r, bl_r, out_ref):
        acc = jnp.dot(x_ref[...], wl2_r[...], preferred_element_type=_F32) + bl_r[...]
        nf = nfa_ref[...] + nfb_ref[...]
        for h in range(_H):
            nh = nf[:, h * _DP:(h + 1) * _DP]
            sden = nh[:, _D:_D + 1]
            inv = jnp.where(sden > 0, 1.0 / sden, 0.0)
            acc = acc + jnp.dot(nh * inv, wl1_r[h], preferred_element_type=_F32)
        out_ref[...] = acc

    weights = [wl1h, wl2, bl]
    in_specs = [
        pl.BlockSpec((bn, _H * _DP), lambda i: (i, 0)),
        pl.BlockSpec((bn, _H * _DP), lambda i: (i, 0)),
        pl.BlockSpec((bn, _D), lambda i: (i, 0)),
    ] + [pl.BlockSpec(w.shape, lambda i, nd=w.ndim: (0,) * nd) for w in weights]

    return pl.pallas_call(
        body,
        grid=(grid,),
        in_specs=in_specs,
        out_specs=pl.BlockSpec((bn, _D), lambda i: (i, 0)),
        out_shape=jax.ShapeDtypeStruct((_N, _D), _F32),
    )(nf_a, nf_b, node_in, *weights)


def kernel(node_in, node_embed, edge_sh, edge_length_embedding, edge_src,
           edge_dst, batch, params):
    p = params
    ele = edge_length_embedding
    src = edge_src.astype(jnp.int32)
    dst = edge_dst.astype(jnp.int32)
    idx_all = jnp.concatenate([src, dst])
    idx2d = jnp.pad(idx_all.reshape(32, 125, _GB),
                    ((0, 0), (0, 3), (0, 0))).reshape(32 * 128, _GB)
    _E1 = 79360
    it1 = _E1 // _NS // _GB
    it2 = (_E - _E1) // _NS // _GB
    dst2d_a = jnp.pad(dst[:_E1].reshape(_NS, it1, _GB),
                      ((0, 0), (0, 64 - it1), (0, 0))).reshape(_NS * 64, _GB)
    dst2d_b = jnp.pad(dst[_E1:].reshape(_NS, it2, _GB),
                      ((0, 0), (0, 64 - it2), (0, 0))).reshape(_NS * 64, _GB)
    row = lambda v: v.reshape(1, -1)

    node_pad = jnp.pad(node_in, ((0, 0), (0, _DP - _D)))
    wpre = p["pre"]["w"]
    padk = lambda w: jnp.pad(w, ((0, _DP - _D), (0, 0)))
    wp1, wp2, wp3 = padk(wpre[:_D]), padk(wpre[_D:2 * _D]), wpre[2 * _D:]
    bp = row(p["pre"]["b"])

    def padh(w3):
        return jnp.pad(w3.transpose(1, 0, 2), ((0, 0), (0, 0), (0, _DP - _D)))

    wtp2 = p["tp2"]["w"]
    wt1 = padh(wtp2[:_D].reshape(_D, _H, _D))
    wt2 = padh(jnp.pad(wtp2[_D:_D + 9], ((0, 7), (0, 0))).reshape(16, _H, _D))
    wt3 = padh(wtp2[_D + 9:].reshape(_DLEN, _H, _D))
    bt = jnp.pad(p["tp2"]["b"].reshape(_H, 1, _D), ((0, 0), (0, 0), (0, _DP - _D)))
    bt = bt.at[:, :, _D].set(1.0)
    lewh = jnp.pad(p["le"]["w"].reshape(_H, _D, 64), ((0, 0), (0, _DP - _D), (0, 0)))

    shp = jnp.pad(edge_sh, ((0, 0), (0, 7)))

    gath = _sc_gather(node_pad, idx2d)
    alpha8, gmax = _tc_alpha(ele, p)
    heavy_weights = [
        wp1, wp2, wp3, bp, wt1, wt2, wt3, bt, lewh, row(p["le"]["b"]),
        p["s1"]["w"], row(p["s1"]["b"]), row(p["s1g"]), row(p["s1b"]),
        p["s2"]["w"], row(p["s2"]["b"]), row(p["s2g"]), row(p["s2b"]),
        p["s3"]["w"], row(p["s3"]["b"]),
    ]
    zeros128 = jnp.zeros((_NP // _NS, 128), _F32)
    pv_a, esc_a = _tc_edges(gath, shp, ele, alpha8, gmax, heavy_weights,
                            0, _E1)
    nf_a = _sc_node_scatter(pv_a, dst2d_a, zeros128, it1)
    pv_b, esc_b = _tc_edges(gath, shp, ele, alpha8, gmax, heavy_weights,
                            _E1, _E - _E1)
    nf_b = _sc_node_scatter(pv_b, dst2d_b, zeros128, it2)
    esc = jnp.concatenate([esc_a, esc_b], axis=0)
    wl = p["lin"]["w"]
    wl1h = jnp.pad(wl[:_H * _D].reshape(_H, _D, _D), ((0, 0), (0, _DP - _D), (0, 0)))
    node_out = _tc_nodes(nf_a, nf_b, node_in, wl1h, wl[_H * _D:], row(p["lin"]["b"]))
    return node_out, esc


# speedup vs baseline: 9.8526x; 1.0910x over previous
import functools

import jax
import jax.numpy as jnp
from jax import lax
from jax.experimental import pallas as pl
from jax.experimental.pallas import tpu as pltpu
from jax.experimental.pallas import tpu_sc as plsc

_N = 10000
_NP = 10240
_E = 160000
_D = 480
_DP = 512
_H = 4
_DLEN = 128

_NC = 2
_NS = 16
_GB = 80

_F32 = jnp.float32


def _ln(x, g, b):
    mu = jnp.mean(x, axis=-1, keepdims=True)
    var = jnp.mean((x - mu) ** 2, axis=-1, keepdims=True)
    return g * (x - mu) * lax.rsqrt(var + 1e-6) + b


def _silu(x):
    return x * jax.nn.sigmoid(x)


def _sc_gather(table, idx2d, blocks):
    mesh = plsc.VectorSubcoreMesh(core_axis_name="c", subcore_axis_name="s")
    last = blocks - 1

    @functools.partial(
        pl.kernel,
        mesh=mesh,
        out_type=jax.ShapeDtypeStruct((32 * blocks * _GB, _DP), _F32),
        scratch_types=[
            pltpu.VMEM((32, _GB), jnp.int32),
            pltpu.VMEM((_GB, _DP), _F32),
            pltpu.VMEM((_GB, _DP), _F32),
            pltpu.SemaphoreType.DMA,
            pltpu.SemaphoreType.DMA,
            pltpu.SemaphoreType.DMA,
            pltpu.SemaphoreType.DMA,
        ],
    )
    def gather_k(tbl_hbm, idx_hbm, out_hbm, idx2, b0, b1, g0, g1, w0, w1):
        c = lax.axis_index("c")
        s = lax.axis_index("s")
        wid = s * _NC + c
        base = wid * blocks * _GB
        pltpu.sync_copy(idx_hbm.at[pl.ds(wid * 32, 32)], idx2)

        def g_start(blk, buf, sem):
            pltpu.async_copy(tbl_hbm.at[idx2.at[blk]], buf, sem)

        def g_wait(blk, buf, sem):
            pltpu.make_async_copy(tbl_hbm.at[idx2.at[blk]], buf, sem).wait()

        def w_start(blk, buf, sem):
            pltpu.async_copy(buf, out_hbm.at[pl.ds(base + blk * _GB, _GB)], sem)

        def w_wait(blk, buf, sem):
            pltpu.make_async_copy(
                buf, out_hbm.at[pl.ds(base + blk * _GB, _GB)], sem).wait()

        g_start(0, b0, g0)
        g_start(1, b1, g1)

        def rnd(r, carry):
            blk0 = 2 * r
            blk1 = 2 * r + 1
            g_wait(blk0, b0, g0)
            w_start(blk0, b0, w0)
            g_wait(blk1, b1, g1)
            w_start(blk1, b1, w1)
            w_wait(blk0, b0, w0)
            g_start(jnp.minimum(blk0 + 2, last), b0, g0)
            w_wait(blk1, b1, w1)
            g_start(jnp.minimum(blk1 + 2, last), b1, g1)
            return carry

        lax.fori_loop(0, blocks // 2, rnd, 0)
        g_wait(last, b0, g0)
        g_wait(last, b1, g1)
        if blocks % 2:
            pltpu.sync_copy(b0, out_hbm.at[pl.ds(base + last * _GB, _GB)])

    return gather_k(table, idx2d)


def _tc_alpha(ele, p):
    be = 2000
    grid = _E // be

    def body(ele_ref, a1w, a1b, a1g, a1bb, a2w, a2b, a2g, a2bb, a3w, a3b,
             alpha_ref, g_ref):
        a = jnp.dot(ele_ref[...], a1w[...], preferred_element_type=_F32) + a1b[...]
        a = _silu(_ln(a, a1g[...], a1bb[...]))
        a = jnp.dot(a, a2w[...], preferred_element_type=_F32) + a2b[...]
        a = _silu(_ln(a, a2g[...], a2bb[...]))
        al8 = jnp.dot(a, a3w[...], preferred_element_type=_F32) + a3b[...]
        alpha_ref[...] = al8
        bm = jnp.max(al8[:, :_H])
        prev = jnp.where(pl.program_id(0) == 0, -3.0e38, g_ref[0, 0])
        g_ref[0, 0] = jnp.maximum(prev, bm)

    row = lambda v: v.reshape(1, -1)
    a3w8 = jnp.pad(p["a3"]["w"], ((0, 0), (0, 8 - _H)))
    a3b8 = row(jnp.pad(p["a3"]["b"], (0, 8 - _H)))
    weights = [
        p["a1"]["w"], row(p["a1"]["b"]), row(p["a1g"]), row(p["a1b"]),
        p["a2"]["w"], row(p["a2"]["b"]), row(p["a2g"]), row(p["a2b"]),
        a3w8, a3b8,
    ]
    in_specs = [pl.BlockSpec((be, _DLEN), lambda i: (i, 0))] + [
        pl.BlockSpec(w.shape, lambda i, nd=w.ndim: (0,) * nd) for w in weights
    ]
    alpha8, gmax = pl.pallas_call(
        body,
        grid=(grid,),
        in_specs=in_specs,
        out_specs=[
            pl.BlockSpec((be, 8), lambda i: (i, 0)),
            pl.BlockSpec((1, 1), lambda i: (0, 0), memory_space=pltpu.SMEM),
        ],
        out_shape=[
            jax.ShapeDtypeStruct((_E, 8), _F32),
            jax.ShapeDtypeStruct((1, 1), _F32),
        ],
    )(ele, *weights)
    return alpha8, gmax


def _tc_edges(xsxd, shp, ele, alpha8, gmax, weights, e_start, e_count):
    be = 640
    grid = e_count // be
    ofs = e_start // be
    xd_ofs = e_count // be

    def body(xs_ref, xd_ref, shp_ref, ele_ref, al_ref, g_ref,
             wp1_r, wp2_r, wp3_r, bp_r, wt1_r, wt2_r, wt3_r, bt_r,
             lew_r, leb_r, s1w_r, s1b_r, s1g_r, s1bb_r,
             s2w_r, s2b_r, s2g_r, s2bb_r, s3w_r, s3b_r,
             pv_ref, esc_ref):
        xs = xs_ref[...]
        xd = xd_ref[...]
        ele_b = ele_ref[...]
        msg = (jnp.dot(xs, wp1_r[...], preferred_element_type=_F32)
               + jnp.dot(xd, wp2_r[...], preferred_element_type=_F32)
               + jnp.dot(ele_b, wp3_r[...], preferred_element_type=_F32)
               + bp_r[...])
        shp_b = shp_ref[...]
        al = al_ref[...]
        pw = jnp.exp(al - g_ref[0, 0])
        es = jnp.zeros((be, 64), _F32) + leb_r[...]
        for h in range(_H):
            vh = (jnp.dot(msg, wt1_r[h], preferred_element_type=_F32)
                  + jnp.dot(shp_b, wt2_r[h], preferred_element_type=_F32)
                  + jnp.dot(ele_b, wt3_r[h], preferred_element_type=_F32)
                  + bt_r[h])
            eo = vh * al[:, h:h + 1]
            es = es + jnp.dot(eo, lew_r[h], preferred_element_type=_F32)
            pv_ref[:, h * _DP:(h + 1) * _DP] = vh * pw[:, h:h + 1]
        es = _silu(_ln(jnp.dot(es, s1w_r[...], preferred_element_type=_F32)
                       + s1b_r[...], s1g_r[...], s1bb_r[...]))
        es = _silu(_ln(jnp.dot(es, s2w_r[...], preferred_element_type=_F32)
                       + s2b_r[...], s2g_r[...], s2bb_r[...]))
        esc_ref[...] = jnp.dot(es, s3w_r[...], preferred_element_type=_F32) + s3b_r[...]

    in_specs = [
        pl.BlockSpec((be, _DP), lambda i: (i, 0)),
        pl.BlockSpec((be, _DP), lambda i, o=xd_ofs: (i + o, 0)),
        pl.BlockSpec((be, 16), lambda i, o=ofs: (i + o, 0)),
        pl.BlockSpec((be, _DLEN), lambda i, o=ofs: (i + o, 0)),
        pl.BlockSpec((be, 8), lambda i, o=ofs: (i + o, 0)),
        pl.BlockSpec((1, 1), lambda i: (0, 0), memory_space=pltpu.SMEM),
    ] + [pl.BlockSpec(w.shape, lambda i, nd=w.ndim: (0,) * nd) for w in weights]

    pv, esc = pl.pallas_call(
        body,
        grid=(grid,),
        in_specs=in_specs,
        out_specs=[
            pl.BlockSpec((be, _H * _DP), lambda i: (i, 0)),
            pl.BlockSpec((be, 32), lambda i: (i, 0)),
        ],
        out_shape=[
            jax.ShapeDtypeStruct((e_count, _H * _DP), _F32),
            jax.ShapeDtypeStruct((e_count, 32), _F32),
        ],
    )(xsxd, xsxd, shp, ele, alpha8, gmax, *weights)
    return pv, esc


def _sc_node_scatter(pv2, dst2d, zeros128, iters):
    mesh = plsc.VectorSubcoreMesh(core_axis_name="c", subcore_axis_name="s")
    e_per_t = iters * _GB
    cw = 128
    chunks_per_sc = (_H * _DP) // cw // _NC
    rows_per_t = _NP // _NS

    nbuf = 3
    last = iters - 1

    @functools.partial(
        pl.kernel,
        mesh=mesh,
        out_type=jax.ShapeDtypeStruct((_NP, _H * _DP), _F32),
        scratch_types=[
            pltpu.VMEM_SHARED((_NP, cw), _F32),
            pltpu.VMEM((32, _GB), jnp.int32),
            [pltpu.VMEM((_GB, cw), _F32) for _ in range(nbuf)],
            [pltpu.SemaphoreType.DMA for _ in range(nbuf)],
            [pltpu.SemaphoreType.DMA for _ in range(nbuf)],
        ],
    )
    def nf_k(pv_hbm, dst_hbm, z_hbm, out_hbm, acc, idx2, bufs, lsems, ssems):
        c = lax.axis_index("c")
        s = lax.axis_index("s")
        r0 = s * rows_per_t
        pltpu.sync_copy(dst_hbm.at[pl.ds(s * 32, 32)], idx2)

        def ld_start(blk, co, b, sem):
            pltpu.async_copy(
                pv_hbm.at[pl.ds(s * e_per_t + blk * _GB, _GB), pl.ds(co, cw)],
                b, sem)

        def ld_wait(blk, co, b, sem):
            pltpu.make_async_copy(
                pv_hbm.at[pl.ds(s * e_per_t + blk * _GB, _GB), pl.ds(co, cw)],
                b, sem).wait()

        def sc_start(blk, b, sem):
            pltpu.async_copy(b, acc.at[idx2.at[blk]], sem, add=True)

        def sc_wait(blk, b, sem):
            pltpu.make_async_copy(b, acc.at[idx2.at[blk]], sem).wait()

        for j in range(chunks_per_sc):
            co = (c * chunks_per_sc + j) * cw
            for b in range(nbuf):
                ld_start(b, co, bufs[b], lsems[b])
            pltpu.sync_copy(z_hbm, acc.at[pl.ds(r0, rows_per_t)])
            plsc.subcore_barrier()

            def rnd(r, carry):
                base_blk = nbuf * r
                for b in range(nbuf):
                    ld_wait(base_blk + b, co, bufs[b], lsems[b])
                    sc_start(base_blk + b, bufs[b], ssems[b])
                for b in range(nbuf):
                    sc_wait(base_blk + b, bufs[b], ssems[b])
                    ld_start(jnp.minimum(base_blk + nbuf + b, last), co,
                             bufs[b], lsems[b])
                return carry

            full_rounds = iters // nbuf
            rem = iters - full_rounds * nbuf
            lax.fori_loop(0, full_rounds, rnd, 0)
            for b in range(nbuf):
                ld_wait(min(full_rounds * nbuf + b, last), co,
                        bufs[b], lsems[b])
            for b in range(rem):
                pltpu.sync_copy(bufs[b],
                                acc.at[idx2.at[full_rounds * nbuf + b]],
                                add=True)
            plsc.subcore_barrier()
            pltpu.sync_copy(acc.at[pl.ds(r0, rows_per_t)],
                            out_hbm.at[pl.ds(r0, rows_per_t), pl.ds(co, cw)])
            plsc.subcore_barrier()

    return nf_k(pv2, dst2d, zeros128)


def _tc_nodes(nfs, node_in, wl1h, wl2, bl):
    bn = 400
    grid = _N // bn
    nq = len(nfs)

    def body(*refs):
        nf_refs = refs[:nq]
        x_ref, wl1_r, wl2_r, bl_r, out_ref = refs[nq:]
        acc = jnp.dot(x_ref[...], wl2_r[...], preferred_element_type=_F32) + bl_r[...]
        nf = nf_refs[0][...]
        for r in nf_refs[1:]:
            nf = nf + r[...]
        for h in range(_H):
            nh = nf[:, h * _DP:(h + 1) * _DP]
            sden = nh[:, _D:_D + 1]
            inv = jnp.where(sden > 0, 1.0 / sden, 0.0)
            acc = acc + jnp.dot(nh * inv, wl1_r[h], preferred_element_type=_F32)
        out_ref[...] = acc

    weights = [wl1h, wl2, bl]
    in_specs = [
        pl.BlockSpec((bn, _H * _DP), lambda i: (i, 0)) for _ in range(nq)
    ] + [
        pl.BlockSpec((bn, _D), lambda i: (i, 0)),
    ] + [pl.BlockSpec(w.shape, lambda i, nd=w.ndim: (0,) * nd) for w in weights]

    return pl.pallas_call(
        body,
        grid=(grid,),
        in_specs=in_specs,
        out_specs=pl.BlockSpec((bn, _D), lambda i: (i, 0)),
        out_shape=jax.ShapeDtypeStruct((_N, _D), _F32),
    )(*nfs, node_in, *weights)


def kernel(node_in, node_embed, edge_sh, edge_length_embedding, edge_src,
           edge_dst, batch, params):
    p = params
    ele = edge_length_embedding
    src = edge_src.astype(jnp.int32)
    dst = edge_dst.astype(jnp.int32)
    idx_all = jnp.concatenate([src, dst])
    bounds = [0, 39680, 79360, 119040, _E]
    qidx, qdst, qit = [], [], []
    for q in range(4):
        qs, qe = bounds[q], bounds[q + 1]
        eq = qe - qs
        gb = (2 * eq) // 32 // _GB
        it = eq // _NS // _GB
        iq = jnp.concatenate([src[qs:qe], dst[qs:qe]]).reshape(32, gb, _GB)
        qidx.append(jnp.pad(iq, ((0, 0), (0, 32 - gb), (0, 0))).reshape(32 * 32, _GB))
        dq = dst[qs:qe].reshape(_NS, it, _GB)
        qdst.append(jnp.pad(dq, ((0, 0), (0, 32 - it), (0, 0))).reshape(_NS * 32, _GB))
        qit.append(it)
    row = lambda v: v.reshape(1, -1)

    node_pad = jnp.pad(node_in, ((0, 0), (0, _DP - _D)))
    wpre = p["pre"]["w"]
    padk = lambda w: jnp.pad(w, ((0, _DP - _D), (0, 0)))
    wp1, wp2, wp3 = padk(wpre[:_D]), padk(wpre[_D:2 * _D]), wpre[2 * _D:]
    bp = row(p["pre"]["b"])

    def padh(w3):
        return jnp.pad(w3.transpose(1, 0, 2), ((0, 0), (0, 0), (0, _DP - _D)))

    wtp2 = p["tp2"]["w"]
    wt1 = padh(wtp2[:_D].reshape(_D, _H, _D))
    wt2 = padh(jnp.pad(wtp2[_D:_D + 9], ((0, 7), (0, 0))).reshape(16, _H, _D))
    wt3 = padh(wtp2[_D + 9:].reshape(_DLEN, _H, _D))
    bt = jnp.pad(p["tp2"]["b"].reshape(_H, 1, _D), ((0, 0), (0, 0), (0, _DP - _D)))
    bt = bt.at[:, :, _D].set(1.0)
    lewh = jnp.pad(p["le"]["w"].reshape(_H, _D, 64), ((0, 0), (0, _DP - _D), (0, 0)))

    shp = jnp.pad(edge_sh, ((0, 0), (0, 7)))

    alpha8, gmax = _tc_alpha(ele, p)
    heavy_weights = [
        wp1, wp2, wp3, bp, wt1, wt2, wt3, bt, lewh, row(p["le"]["b"]),
        p["s1"]["w"], row(p["s1"]["b"]), row(p["s1g"]), row(p["s1b"]),
        p["s2"]["w"], row(p["s2"]["b"]), row(p["s2g"]), row(p["s2b"]),
        p["s3"]["w"], row(p["s3"]["b"]),
    ]
    zeros128 = jnp.zeros((_NP // _NS, 128), _F32)
    nfs, escs = [], []
    for q in range(4):
        qs, qe = bounds[q], bounds[q + 1]
        gath_q = _sc_gather(node_pad, qidx[q], (2 * (qe - qs)) // 32 // _GB)
        pv_q, esc_q = _tc_edges(gath_q, shp, ele, alpha8, gmax,
                                heavy_weights, qs, qe - qs)
        nfs.append(_sc_node_scatter(pv_q, qdst[q], zeros128, qit[q]))
        escs.append(esc_q)
    esc = jnp.concatenate(escs, axis=0)
    wl = p["lin"]["w"]
    wl1h = jnp.pad(wl[:_H * _D].reshape(_H, _D, _D), ((0, 0), (0, _DP - _D), (0, 0)))
    node_out = _tc_nodes(nfs, node_in, wl1h, wl[_H * _D:], row(p["lin"]["b"]))
    return node_out, esc
